# Initial kernel scaffold; baseline (speedup 1.0000x reference)
import jax
import jax.numpy as jnp
from jax.experimental import pallas as pl


def kernel(indices, weight):
    raise NotImplementedError("write your pallas kernel here")


# speedup vs baseline: 1.0232x; 1.0232x over previous
import functools

import jax
import jax.numpy as jnp
from jax import lax
from jax.experimental import pallas as pl
from jax.experimental.pallas import tpu as pltpu
from jax.experimental.pallas import tpu_sc as plsc

ROWS_PER_GATHER = 128
NUM_CORES = 2
NUM_SUBCORES = 16
NUM_WORKERS = NUM_CORES * NUM_SUBCORES


@functools.cache
def _make_gather(num_groups_total: int, dim: int):
    gpw = num_groups_total // NUM_WORKERS
    mesh = plsc.VectorSubcoreMesh(core_axis_name="c", subcore_axis_name="s")

    @functools.partial(
        pl.kernel,
        mesh=mesh,
        out_type=jax.ShapeDtypeStruct(
            (num_groups_total * ROWS_PER_GATHER, dim), jnp.float32
        ),
        scratch_types=[
            pltpu.VMEM((gpw, ROWS_PER_GATHER), jnp.int32),
            pltpu.VMEM((ROWS_PER_GATHER, dim), jnp.float32),
            pltpu.SemaphoreType.DMA,
        ],
        compiler_params=pltpu.CompilerParams(use_tc_tiling_on_sc=False),
    )
    def gather_kernel(idx_hbm, table_hbm, out_hbm, idx_v, rows_v, sem):
        wid = lax.axis_index("s") * NUM_CORES + lax.axis_index("c")
        g0 = wid * gpw
        pltpu.sync_copy(idx_hbm.at[pl.ds(g0, gpw)], idx_v)

        def body(g, carry):
            pltpu.async_copy(table_hbm.at[idx_v.at[g]], rows_v, sem).wait()
            pltpu.sync_copy(
                rows_v,
                out_hbm.at[pl.ds((g0 + g) * ROWS_PER_GATHER, ROWS_PER_GATHER)],
            )
            return carry

        lax.fori_loop(0, gpw, body, 0)

    return gather_kernel


def kernel(indices, weight):
    b = indices.size
    idx2d = indices.reshape(b // ROWS_PER_GATHER, ROWS_PER_GATHER).astype(jnp.int32)
    out = _make_gather(idx2d.shape[0], weight.shape[1])(idx2d, weight)
    return out.reshape(indices.shape + (weight.shape[1],))


# speedup vs baseline: 1.1110x; 1.0858x over previous
import functools

import jax
import jax.numpy as jnp
from jax import lax
from jax.experimental import pallas as pl
from jax.experimental.pallas import tpu as pltpu
from jax.experimental.pallas import tpu_sc as plsc

ROWS_PER_GATHER = 128
KG = 10
NUM_CORES = 2
NUM_SUBCORES = 16
NUM_WORKERS = NUM_CORES * NUM_SUBCORES


@functools.cache
def _make_gather(num_groups_total: int, dim: int):
    gpw = num_groups_total // NUM_WORKERS
    blocks = gpw // KG
    assert blocks % 2 == 0 and blocks >= 4
    block_rows = KG * ROWS_PER_GATHER
    mesh = plsc.VectorSubcoreMesh(core_axis_name="c", subcore_axis_name="s")

    @functools.partial(
        pl.kernel,
        mesh=mesh,
        out_type=jax.ShapeDtypeStruct(
            (num_groups_total * ROWS_PER_GATHER, dim), jnp.float32
        ),
        scratch_types=[
            pltpu.VMEM((gpw, ROWS_PER_GATHER), jnp.int32),
            pltpu.VMEM((block_rows, dim), jnp.float32),
            pltpu.VMEM((block_rows, dim), jnp.float32),
            pltpu.SemaphoreType.DMA,
            pltpu.SemaphoreType.DMA,
            pltpu.SemaphoreType.DMA,
            pltpu.SemaphoreType.DMA,
        ],
        compiler_params=pltpu.CompilerParams(use_tc_tiling_on_sc=False),
    )
    def gather_kernel(
        idx_hbm, table_hbm, out_hbm, idx_v, rows_a, rows_b, sga, sgb, soa, sob
    ):
        wid = lax.axis_index("s") * NUM_CORES + lax.axis_index("c")
        g0 = wid * gpw
        pltpu.sync_copy(idx_hbm.at[pl.ds(g0, gpw)], idx_v)

        def fire(blk, rows_v, sem):
            for j in range(KG):
                pltpu.async_copy(
                    table_hbm.at[idx_v.at[blk * KG + j]],
                    rows_v.at[pl.ds(j * ROWS_PER_GATHER, ROWS_PER_GATHER)],
                    sem,
                )

        def drain_gathers(rows_v, sem):
            for j in range(KG):
                pltpu.make_async_copy(
                    table_hbm.at[idx_v.at[j]],
                    rows_v.at[pl.ds(j * ROWS_PER_GATHER, ROWS_PER_GATHER)],
                    sem,
                ).wait()

        def copy_out(blk, rows_v, sem):
            return pltpu.async_copy(
                rows_v, out_hbm.at[pl.ds((g0 + blk * KG) * ROWS_PER_GATHER, block_rows)], sem
            )

        def drain_out(blk, rows_v, sem):
            pltpu.make_async_copy(
                rows_v, out_hbm.at[pl.ds((g0 + blk * KG) * ROWS_PER_GATHER, block_rows)], sem
            ).wait()

        fire(0, rows_a, sga)
        drain_gathers(rows_a, sga)
        copy_out(0, rows_a, soa)
        fire(1, rows_b, sgb)

        def body(ii, carry):
            b1 = 2 * ii + 1
            b2 = 2 * ii + 2
            drain_gathers(rows_b, sgb)
            drain_out(b2 - 2, rows_a, soa)
            fire(b2, rows_a, sga)
            copy_out(b1, rows_b, sob)
            drain_gathers(rows_a, sga)
            drain_out(b1, rows_b, sob)
            fire(b2 + 1, rows_b, sgb)
            copy_out(b2, rows_a, soa)
            return carry

        lax.fori_loop(0, blocks // 2 - 1, body, 0)

        drain_gathers(rows_b, sgb)
        drain_out(blocks - 2, rows_a, soa)
        copy_out(blocks - 1, rows_b, sob)
        drain_out(blocks - 1, rows_b, sob)

    return gather_kernel


def kernel(indices, weight):
    b = indices.size
    idx2d = indices.reshape(b // ROWS_PER_GATHER, ROWS_PER_GATHER).astype(jnp.int32)
    out = _make_gather(idx2d.shape[0], weight.shape[1])(idx2d, weight)
    return out.reshape(indices.shape + (weight.shape[1],))
